# baseline (speedup 1.0000x reference)
import functools

import jax
import jax.numpy as jnp
from jax import lax
from jax.experimental import pallas as pl
from jax.experimental.pallas import tpu as pltpu

CH = 500
NB = 4
NSTEPS = 20


def _body(h_any, nei_any, wt_ref, b_ref, out_any, nbuf, hbuf, obuf, nsem, hsem, osem,
          *, inv_count):
    def nei_copy(step, k):
        return pltpu.make_async_copy(
            nei_any.at[pl.ds(step * CH, CH)], nbuf.at[k], nsem.at[k]
        )

    def h_copy(step, k):
        return pltpu.make_async_copy(
            h_any.at[pl.ds(step * CH, CH)], hbuf.at[k], hsem.at[k]
        )

    def out_copy(step, k):
        return pltpu.make_async_copy(
            obuf.at[k], out_any.at[pl.ds(step * CH, CH)], osem.at[k]
        )

    for k in range(NB):
        nei_copy(k, k).start()
        h_copy(k, k).start()

    def loop_body(it, _):
        s = it * NB
        for k in range(NB):
            step = s + k
            nei_copy(step, k).wait()
            h_copy(step, k).wait()

            @pl.when(it >= 1)
            def _():
                out_copy(step - NB, k).wait()

            agg = (jnp.sum(nbuf[k], axis=1) + hbuf[k]) * inv_count
            obuf[k] = (
                jnp.dot(agg, wt_ref[...], preferred_element_type=jnp.float32)
                + b_ref[...]
            )
            out_copy(step, k).start()

            @pl.when(it < NSTEPS // NB - 1)
            def _():
                nei_copy(step + NB, k).start()
                h_copy(step + NB, k).start()

        return 0

    lax.fori_loop(0, NSTEPS // NB, loop_body, 0)

    for k in range(NB):
        out_copy(NSTEPS - NB + k, k).wait()


@jax.jit
def kernel(h, nei, W, b):
    n, in_feats = h.shape
    deg = nei.shape[1]
    out_feats = W.shape[0]

    wt = W.T
    b2 = b.reshape(1, out_feats)

    body = functools.partial(_body, inv_count=float(1.0 / (deg + 1)))

    return pl.pallas_call(
        body,
        in_specs=[
            pl.BlockSpec(memory_space=pl.ANY),
            pl.BlockSpec(memory_space=pl.ANY),
            pl.BlockSpec(memory_space=pltpu.MemorySpace.VMEM),
            pl.BlockSpec(memory_space=pltpu.MemorySpace.VMEM),
        ],
        out_specs=pl.BlockSpec(memory_space=pl.ANY),
        out_shape=jax.ShapeDtypeStruct((n, out_feats), jnp.float32),
        scratch_shapes=[
            pltpu.VMEM((NB, CH, deg, in_feats), jnp.float32),
            pltpu.VMEM((NB, CH, in_feats), jnp.float32),
            pltpu.VMEM((NB, CH, out_feats), jnp.float32),
            pltpu.SemaphoreType.DMA((NB,)),
            pltpu.SemaphoreType.DMA((NB,)),
            pltpu.SemaphoreType.DMA((NB,)),
        ],
    )(h, nei, wt, b2)


# speedup vs baseline: 1.0717x; 1.0717x over previous
import functools

import jax
import jax.numpy as jnp
from jax.experimental import pallas as pl


def _agg_body(h_ref, nei_ref, wt_ref, b_ref, out_ref, *, inv_count):
    s = jnp.sum(nei_ref[...], axis=1) + h_ref[...]
    agg = s * inv_count
    out_ref[...] = (
        jnp.dot(agg, wt_ref[...], preferred_element_type=jnp.float32) + b_ref[...]
    )


@jax.jit
def kernel(h, nei, W, b):
    n, in_feats = h.shape
    deg = nei.shape[1]
    out_feats = W.shape[0]

    block = 448
    grid = (pl.cdiv(n, block),)

    wt = W.T
    b2 = b.reshape(1, out_feats)

    body = functools.partial(_agg_body, inv_count=float(1.0 / (deg + 1)))

    return pl.pallas_call(
        body,
        grid=grid,
        in_specs=[
            pl.BlockSpec((block, in_feats), lambda i: (i, 0)),
            pl.BlockSpec((block, deg, in_feats), lambda i: (i, 0, 0)),
            pl.BlockSpec((in_feats, out_feats), lambda i: (0, 0)),
            pl.BlockSpec((1, out_feats), lambda i: (0, 0)),
        ],
        out_specs=pl.BlockSpec((block, out_feats), lambda i: (i, 0)),
        out_shape=jax.ShapeDtypeStruct((n, out_feats), jnp.float32),
    )(h, nei, wt, b2)
